# Initial kernel scaffold; baseline (speedup 1.0000x reference)
import jax
import jax.numpy as jnp
from jax.experimental import pallas as pl


def kernel(input, target, batch_idx):
    raise NotImplementedError("write your pallas kernel here")


# speedup vs baseline: 6.6423x; 6.6423x over previous
import functools

import jax
import jax.numpy as jnp
from jax.experimental import pallas as pl
from jax.experimental.pallas import tpu as pltpu

NUM_SEG = 16
TOTAL_TOK = 32768
D = 1024
BT = 2048
NB = TOTAL_TOK // BT


def _body(idx_ref, x_ref, t_ref, o_ref, acc_d, acc_t):
    i = pl.program_id(0)

    idx = idx_ref[0, 0, :]
    onehot = (jax.lax.broadcasted_iota(jnp.int32, (NUM_SEG, BT), 0)
              == idx[None, :]).astype(jnp.float32)

    x = x_ref[...]
    t = t_ref[...]
    d = x - t
    ds = d * d
    ts = t * t
    pd = jnp.dot(onehot, ds, preferred_element_type=jnp.float32)
    pt = jnp.dot(onehot, ts, preferred_element_type=jnp.float32)

    @pl.when(i == 0)
    def _init():
        acc_d[...] = pd
        acc_t[...] = pt

    @pl.when(i > 0)
    def _accum():
        acc_d[...] += pd
        acc_t[...] += pt

    @pl.when(i == NB - 1)
    def _fin():
        dn = jnp.sqrt(acc_d[...])
        tn = jnp.maximum(jnp.sqrt(acc_t[...]), 1.0)
        o_ref[0, 0] = jnp.mean(dn / tn)


@jax.jit
def _run(inp, tgt, idx3):
    out = pl.pallas_call(
        _body,
        grid=(NB,),
        in_specs=[
            pl.BlockSpec((1, 1, BT), lambda i: (i, 0, 0)),
            pl.BlockSpec((BT, D), lambda i: (i, 0)),
            pl.BlockSpec((BT, D), lambda i: (i, 0)),
        ],
        out_specs=pl.BlockSpec((1, 1), lambda i: (0, 0), memory_space=pltpu.SMEM),
        out_shape=jax.ShapeDtypeStruct((1, 1), jnp.float32),
        scratch_shapes=[
            pltpu.VMEM((NUM_SEG, D), jnp.float32),
            pltpu.VMEM((NUM_SEG, D), jnp.float32),
        ],
    )(idx3, inp, tgt)
    return out[0, 0]


def kernel(input, target, batch_idx):
    idx3 = batch_idx.astype(jnp.int32).reshape(NB, 1, BT)
    return _run(input, target, idx3)
